# baseline (speedup 1.0000x reference)
import functools

import jax
import jax.numpy as jnp
from jax import lax
from jax.experimental import pallas as pl
from jax.experimental.pallas import tpu as pltpu
from jax.experimental.pallas import tpu_sc as plsc

N_CORES = 2
N_SUBCORES = 16
LANES = 16
CHUNK = 128
D = 128
DH = 64


def _sc_scatter_layer(n_src2, n_pad, T, W):
    rpt = n_pad // N_SUBCORES
    mesh = plsc.VectorSubcoreMesh(core_axis_name="c", subcore_axis_name="s")

    @functools.partial(
        pl.kernel,
        out_type=jax.ShapeDtypeStruct((N_CORES * n_pad, W), jnp.float32),
        mesh=mesh,
        scratch_types=[
            pltpu.VMEM((2, CHUNK), jnp.int32),
            pltpu.VMEM((2, CHUNK), jnp.int32),
            pltpu.VMEM((2, CHUNK), jnp.int32),
            pltpu.VMEM((2, CHUNK), jnp.int32),
            pltpu.VMEM((CHUNK, W), jnp.float32),
            pltpu.VMEM((CHUNK, W), jnp.float32),
            pltpu.VMEM_SHARED((n_pad, W), jnp.float32),
            pltpu.SemaphoreType.DMA,
            pltpu.SemaphoreType.DMA,
            pltpu.SemaphoreType.DMA,
            pltpu.SemaphoreType.DMA,
            pltpu.SemaphoreType.DMA,
            pltpu.SemaphoreType.DMA,
        ],
        compiler_params=pltpu.CompilerParams(use_tc_tiling_on_sc=False),
    )
    def k(h2_hbm, idx_hbm, zeros_hbm, out_hbm,
          ib0, ib1, ib2, ib3, rows0, rows1, acc_sh,
          semg0, semg1, si0, si1, si2, si3):
        c = lax.axis_index("c")
        s = lax.axis_index("s")
        ibs = (ib0, ib1, ib2, ib3)
        sis = (si0, si1, si2, si3)

        pltpu.sync_copy(idx_hbm.at[c, s, 0], ib0)
        pltpu.sync_copy(idx_hbm.at[c, s, 1], ib1)
        pltpu.sync_copy(idx_hbm.at[c, s, 2], ib2)
        pltpu.sync_copy(idx_hbm.at[c, s, 3], ib3)
        pltpu.sync_copy(zeros_hbm.at[pl.ds(s * rpt, rpt)],
                        acc_sh.at[pl.ds(s * rpt, rpt)])
        plsc.subcore_barrier()

        pltpu.async_copy(h2_hbm.at[ib0.at[0]], rows0, semg0)
        pltpu.async_copy(h2_hbm.at[ib1.at[0]], rows1, semg1)

        def step(i, kk):
            rows, semg = (rows0, semg0) if kk % 2 == 0 else (rows1, semg1)
            ib = ibs[kk]
            pltpu.make_async_copy(h2_hbm.at[ib.at[0]], rows, semg).wait()
            pltpu.sync_copy(rows, acc_sh.at[ib.at[1]], add=True)

            @pl.when(i + 4 < T)
            def _():
                pltpu.async_copy(idx_hbm.at[c, s, i + 4], ib, sis[kk])

            kn = (kk + 2) % 4
            ibn = ibs[kn]

            @pl.when((i + 2 < T) & (i >= 2))
            def _():
                pltpu.make_async_copy(idx_hbm.at[c, s, i + 2], ibn,
                                      sis[kn]).wait()

            @pl.when(i + 2 < T)
            def _():
                pltpu.async_copy(h2_hbm.at[ibn.at[0]], rows, semg)

        def body(g, carry):
            for kk in range(4):
                step(4 * g + kk, kk)
            return carry

        lax.fori_loop(0, T // 4, body, 0)
        plsc.subcore_barrier()

        pltpu.sync_copy(acc_sh.at[pl.ds(s * rpt, rpt)],
                        out_hbm.at[pl.ds(c * n_pad + s * rpt, rpt)])

    return k


def _tc_z_stats(agg2, h, W_rel2, W_root, b2, n_tgt, W, R=1000):
    nt = n_tgt // R

    def body(agg_ref, h_ref, wr_ref, wroot_ref, b_ref, z_ref, st_ref, acc_ref):
        i = pl.program_id(0)
        agg = agg_ref[...]
        if W == D:
            zr = jnp.dot(agg[0] + agg[1], wr_ref[0],
                         preferred_element_type=jnp.float32)
        else:
            zr = (jnp.dot(agg[0], wr_ref[0],
                          preferred_element_type=jnp.float32)
                  + jnp.dot(agg[1], wr_ref[1],
                            preferred_element_type=jnp.float32))
        z = (zr
             + jnp.dot(h_ref[...], wroot_ref[...],
                       preferred_element_type=jnp.float32)
             + b_ref[...])
        z_ref[...] = z

        @pl.when(i == 0)
        def _():
            acc_ref[...] = jnp.zeros_like(acc_ref)

        acc_ref[0:1, :] += jnp.sum(z, axis=0, keepdims=True)
        acc_ref[1:2, :] += jnp.sum(z * z, axis=0, keepdims=True)
        st_ref[...] = acc_ref[...]

    return pl.pallas_call(
        body,
        grid=(nt,),
        in_specs=[
            pl.BlockSpec((2, R, W), lambda i: (0, i, 0)),
            pl.BlockSpec((R, D), lambda i: (i, 0)),
            pl.BlockSpec((2, W, D), lambda i: (0, 0, 0)),
            pl.BlockSpec((D, D), lambda i: (0, 0)),
            pl.BlockSpec((1, D), lambda i: (0, 0)),
        ],
        out_specs=[
            pl.BlockSpec((R, D), lambda i: (i, 0)),
            pl.BlockSpec((8, D), lambda i: (0, 0)),
        ],
        out_shape=[
            jax.ShapeDtypeStruct((n_tgt, D), jnp.float32),
            jax.ShapeDtypeStruct((8, D), jnp.float32),
        ],
        scratch_shapes=[pltpu.VMEM((8, D), jnp.float32)],
    )(agg2, h, W_rel2, W_root, b2)


def _tc_bn_relu(z, stats, gamma2, beta2, n_tgt, R=1000):
    nt = n_tgt // R
    inv_n = 1.0 / n_tgt

    def body(z_ref, st_ref, g_ref, be_ref, o_ref):
        mu = st_ref[0:1, :] * inv_n
        var = st_ref[1:2, :] * inv_n - mu * mu
        r = lax.rsqrt(var + 1e-5)
        o_ref[...] = jnp.maximum(
            g_ref[...] * (z_ref[...] - mu) * r + be_ref[...], 0.0)

    return pl.pallas_call(
        body,
        grid=(nt,),
        in_specs=[
            pl.BlockSpec((R, D), lambda i: (i, 0)),
            pl.BlockSpec((8, D), lambda i: (0, 0)),
            pl.BlockSpec((1, D), lambda i: (0, 0)),
            pl.BlockSpec((1, D), lambda i: (0, 0)),
        ],
        out_specs=pl.BlockSpec((R, D), lambda i: (i, 0)),
        out_shape=jax.ShapeDtypeStruct((n_tgt, D), jnp.float32),
    )(z, stats, gamma2, beta2)


def _tc_head(h, lin1_W, lin1_b2, lin2_W, lin2_b2, n):
    def body(h_ref, w1_ref, b1_ref, w2_ref, b2_ref, o_ref):
        t = jnp.maximum(
            jnp.dot(h_ref[...], w1_ref[...],
                    preferred_element_type=jnp.float32) + b1_ref[...], 0.0)
        o_ref[...] = (jnp.dot(t, w2_ref[...],
                              preferred_element_type=jnp.float32)
                      + b2_ref[...])

    return pl.pallas_call(
        body,
        out_shape=jax.ShapeDtypeStruct((n, 40), jnp.float32),
    )(h, lin1_W, lin1_b2, lin2_W, lin2_b2)


_LAYERS = [
    dict(E=400000, n_tgt=25000, T=196, n_pad=25088, mode="col"),
    dict(E=160000, n_tgt=10000, T=40, n_pad=10240, mode="edge"),
    dict(E=80000, n_tgt=5000, T=20, n_pad=5120, mode="edge"),
]


def kernel(x, edge_index_0, edge_index_1, edge_index_2,
           W_rel0, W_root0, b0, gamma0, beta0,
           W_rel1, W_root1, b1, gamma1, beta1,
           W_rel2, W_root2, b2, gamma2, beta2,
           lin1_W, lin1_b, lin2_W, lin2_b):
    eis = [edge_index_0, edge_index_1, edge_index_2]
    params = [(W_rel0, W_root0, b0, gamma0, beta0),
              (W_rel1, W_root1, b1, gamma1, beta1),
              (W_rel2, W_root2, b2, gamma2, beta2)]

    h = x
    for li, cfg in enumerate(_LAYERS):
        E, n_tgt, T, n_pad = cfg["E"], cfg["n_tgt"], cfg["T"], cfg["n_pad"]
        W_rel, W_root, b, gamma, beta = params[li]
        ei = eis[li]

        src = ei[0].astype(jnp.int32)
        dst = ei[1].astype(jnp.int32)
        if cfg["mode"] == "col":
            E_ptot = N_SUBCORES * T * CHUNK
            pad = E_ptot - E
            src_p = jnp.concatenate([src, jnp.zeros((pad,), jnp.int32)])
            dst_p = jnp.concatenate([dst, jnp.full((pad,), n_tgt, jnp.int32)])
            src2 = jnp.stack([2 * src_p, 2 * src_p + 1])
            src2 = src2.reshape(2, N_SUBCORES, T, 1, CHUNK)
            dst4 = jnp.broadcast_to(
                dst_p.reshape(1, N_SUBCORES, T, 1, CHUNK), src2.shape)
            W = DH
            table = h.reshape(-1, DH)
            W_rel2 = W_rel.reshape(2, DH, D)
        else:
            E_ptot = N_CORES * N_SUBCORES * T * CHUNK
            pad = E_ptot - E
            src_p = jnp.concatenate([src, jnp.zeros((pad,), jnp.int32)])
            dst_p = jnp.concatenate([dst, jnp.full((pad,), n_tgt, jnp.int32)])
            src2 = src_p.reshape(2, N_SUBCORES, T, 1, CHUNK)
            dst4 = dst_p.reshape(2, N_SUBCORES, T, 1, CHUNK)
            W = D
            table = h
            W_rel2 = jnp.stack([W_rel, W_rel])
        idx = jnp.concatenate([src2, dst4], axis=3)
        zeros = jnp.zeros((n_pad, W), jnp.float32)

        agg_flat = _sc_scatter_layer(table.shape[0], n_pad, T, W)(
            table, idx, zeros)
        agg2 = agg_flat.reshape(N_CORES, n_pad, W)

        z, stats = _tc_z_stats(agg2, h, W_rel2, W_root,
                               b.reshape(1, D), n_tgt, W)
        h = _tc_bn_relu(z, stats, gamma.reshape(1, D), beta.reshape(1, D),
                        n_tgt)

    return _tc_head(h, lin1_W, lin1_b.reshape(1, 64), lin2_W,
                    lin2_b.reshape(1, 40), h.shape[0])


# speedup vs baseline: 1.2093x; 1.2093x over previous
import functools

import jax
import jax.numpy as jnp
from jax import lax
from jax.experimental import pallas as pl
from jax.experimental.pallas import tpu as pltpu
from jax.experimental.pallas import tpu_sc as plsc

N_CORES = 2
N_SUBCORES = 16
LANES = 16
CHUNK = 128
D = 128
DH = 64


def _sc_scatter_layer(n_src2, n_pad, T, W):
    rpt = n_pad // N_SUBCORES
    mesh = plsc.VectorSubcoreMesh(core_axis_name="c", subcore_axis_name="s")

    @functools.partial(
        pl.kernel,
        out_type=jax.ShapeDtypeStruct((N_CORES * n_pad, W), jnp.float32),
        mesh=mesh,
        scratch_types=[
            pltpu.VMEM((2, CHUNK), jnp.int32),
            pltpu.VMEM((2, CHUNK), jnp.int32),
            pltpu.VMEM((2, CHUNK), jnp.int32),
            pltpu.VMEM((2, CHUNK), jnp.int32),
            pltpu.VMEM((CHUNK, W), jnp.float32),
            pltpu.VMEM((CHUNK, W), jnp.float32),
            pltpu.VMEM_SHARED((n_pad, W), jnp.float32),
            pltpu.SemaphoreType.DMA,
            pltpu.SemaphoreType.DMA,
            pltpu.SemaphoreType.DMA,
            pltpu.SemaphoreType.DMA,
            pltpu.SemaphoreType.DMA,
            pltpu.SemaphoreType.DMA,
        ],
        compiler_params=pltpu.CompilerParams(use_tc_tiling_on_sc=False),
    )
    def k(h2_hbm, idx_hbm, zeros_hbm, out_hbm,
          ib0, ib1, ib2, ib3, rows0, rows1, acc_sh,
          semg0, semg1, si0, si1, si2, si3):
        c = lax.axis_index("c")
        s = lax.axis_index("s")
        ibs = (ib0, ib1, ib2, ib3)
        sis = (si0, si1, si2, si3)

        pltpu.sync_copy(idx_hbm.at[c, s, 0], ib0)
        pltpu.sync_copy(idx_hbm.at[c, s, 1], ib1)
        pltpu.sync_copy(idx_hbm.at[c, s, 2], ib2)
        pltpu.sync_copy(idx_hbm.at[c, s, 3], ib3)
        pltpu.sync_copy(zeros_hbm.at[pl.ds(s * rpt, rpt)],
                        acc_sh.at[pl.ds(s * rpt, rpt)])
        plsc.subcore_barrier()

        pltpu.async_copy(h2_hbm.at[ib0.at[0]], rows0, semg0)
        pltpu.async_copy(h2_hbm.at[ib1.at[0]], rows1, semg1)

        def step(i, kk):
            rows, semg = (rows0, semg0) if kk % 2 == 0 else (rows1, semg1)
            ib = ibs[kk]
            pltpu.make_async_copy(h2_hbm.at[ib.at[0]], rows, semg).wait()
            pltpu.sync_copy(rows, acc_sh.at[ib.at[1]], add=True)

            @pl.when(i + 4 < T)
            def _():
                pltpu.async_copy(idx_hbm.at[c, s, i + 4], ib, sis[kk])

            kn = (kk + 2) % 4
            ibn = ibs[kn]

            @pl.when((i + 2 < T) & (i >= 2))
            def _():
                pltpu.make_async_copy(idx_hbm.at[c, s, i + 2], ibn,
                                      sis[kn]).wait()

            @pl.when(i + 2 < T)
            def _():
                pltpu.async_copy(h2_hbm.at[ibn.at[0]], rows, semg)

        def body(g, carry):
            for kk in range(4):
                step(4 * g + kk, kk)
            return carry

        lax.fori_loop(0, T // 4, body, 0)
        plsc.subcore_barrier()

        pltpu.sync_copy(acc_sh.at[pl.ds(s * rpt, rpt)],
                        out_hbm.at[pl.ds(c * n_pad + s * rpt, rpt)])

    return k


def _tc_layer(agg2, h, W_rel2, W_root, b2, gamma2, beta2, n_tgt, W,
              head=None, R=1000):
    nt = n_tgt // R
    inv_n = 1.0 / n_tgt
    d_out = 40 if head is not None else D

    def body(agg_ref, h_ref, wr_ref, wroot_ref, b_ref, g_ref, be_ref,
             *rest):
        if head is not None:
            w1_ref, b1_ref, w2_ref, bh2_ref, o_ref, zbuf, acc = rest
        else:
            o_ref, zbuf, acc = rest
        p = pl.program_id(0)
        i = pl.program_id(1)

        @pl.when(p == 0)
        def _():
            agg = agg_ref[...]
            if W == D:
                zr = jnp.dot(agg[0] + agg[1], wr_ref[0],
                             preferred_element_type=jnp.float32)
            else:
                zr = (jnp.dot(agg[0], wr_ref[0],
                              preferred_element_type=jnp.float32)
                      + jnp.dot(agg[1], wr_ref[1],
                                preferred_element_type=jnp.float32))
            z = (zr
                 + jnp.dot(h_ref[...], wroot_ref[...],
                           preferred_element_type=jnp.float32)
                 + b_ref[...])
            zbuf[pl.ds(i * R, R), :] = z

            @pl.when(i == 0)
            def _():
                acc[...] = jnp.zeros_like(acc)

            acc[0:1, :] += jnp.sum(z, axis=0, keepdims=True)
            acc[1:2, :] += jnp.sum(z * z, axis=0, keepdims=True)

        @pl.when(p == 1)
        def _():
            mu = acc[0:1, :] * inv_n
            var = acc[1:2, :] * inv_n - mu * mu
            r = lax.rsqrt(var + 1e-5)
            z = zbuf[pl.ds(i * R, R), :]
            hh = jnp.maximum(g_ref[...] * (z - mu) * r + be_ref[...], 0.0)
            if head is not None:
                t = jnp.maximum(
                    jnp.dot(hh, w1_ref[...],
                            preferred_element_type=jnp.float32)
                    + b1_ref[...], 0.0)
                o_ref[...] = (jnp.dot(t, w2_ref[...],
                                      preferred_element_type=jnp.float32)
                              + bh2_ref[...])
            else:
                o_ref[...] = hh

    const = lambda p, i: (0, 0)
    in_specs = [
        pl.BlockSpec((2, R, W), lambda p, i: (0, i * (1 - p), 0)),
        pl.BlockSpec((R, D), lambda p, i: (i * (1 - p), 0)),
        pl.BlockSpec((2, W, D), lambda p, i: (0, 0, 0)),
        pl.BlockSpec((D, D), const),
        pl.BlockSpec((1, D), const),
        pl.BlockSpec((1, D), const),
        pl.BlockSpec((1, D), const),
    ]
    args = [agg2, h, W_rel2, W_root, b2, gamma2, beta2]
    if head is not None:
        lin1_W, lin1_b2, lin2_W, lin2_b2 = head
        in_specs += [
            pl.BlockSpec((D, 64), const),
            pl.BlockSpec((1, 64), const),
            pl.BlockSpec((64, 40), const),
            pl.BlockSpec((1, 40), const),
        ]
        args += [lin1_W, lin1_b2, lin2_W, lin2_b2]

    return pl.pallas_call(
        body,
        grid=(2, nt),
        in_specs=in_specs,
        out_specs=pl.BlockSpec((R, d_out), lambda p, i: (i * p, 0)),
        out_shape=jax.ShapeDtypeStruct((n_tgt, d_out), jnp.float32),
        scratch_shapes=[pltpu.VMEM((n_tgt, D), jnp.float32),
                        pltpu.VMEM((8, D), jnp.float32)],
    )(*args)


_LAYERS = [
    dict(E=400000, n_tgt=25000, T=196, n_pad=25088, mode="col"),
    dict(E=160000, n_tgt=10000, T=80, n_pad=10240, mode="col"),
    dict(E=80000, n_tgt=5000, T=40, n_pad=5120, mode="col"),
]


def kernel(x, edge_index_0, edge_index_1, edge_index_2,
           W_rel0, W_root0, b0, gamma0, beta0,
           W_rel1, W_root1, b1, gamma1, beta1,
           W_rel2, W_root2, b2, gamma2, beta2,
           lin1_W, lin1_b, lin2_W, lin2_b):
    eis = [edge_index_0, edge_index_1, edge_index_2]
    params = [(W_rel0, W_root0, b0, gamma0, beta0),
              (W_rel1, W_root1, b1, gamma1, beta1),
              (W_rel2, W_root2, b2, gamma2, beta2)]

    h = x
    for li, cfg in enumerate(_LAYERS):
        E, n_tgt, T, n_pad = cfg["E"], cfg["n_tgt"], cfg["T"], cfg["n_pad"]
        W_rel, W_root, b, gamma, beta = params[li]
        ei = eis[li]

        src = ei[0].astype(jnp.int32)
        dst = ei[1].astype(jnp.int32)
        if cfg["mode"] == "col":
            E_ptot = N_SUBCORES * T * CHUNK
            pad = E_ptot - E
            src_p = jnp.concatenate([src, jnp.zeros((pad,), jnp.int32)])
            dst_p = jnp.concatenate([dst, jnp.full((pad,), n_tgt, jnp.int32)])
            src2 = jnp.stack([2 * src_p, 2 * src_p + 1])
            src2 = src2.reshape(2, N_SUBCORES, T, 1, CHUNK)
            dst4 = jnp.broadcast_to(
                dst_p.reshape(1, N_SUBCORES, T, 1, CHUNK), src2.shape)
            W = DH
            table = h.reshape(-1, DH)
            W_rel2 = W_rel.reshape(2, DH, D)
        else:
            E_ptot = N_CORES * N_SUBCORES * T * CHUNK
            pad = E_ptot - E
            src_p = jnp.concatenate([src, jnp.zeros((pad,), jnp.int32)])
            dst_p = jnp.concatenate([dst, jnp.full((pad,), n_tgt, jnp.int32)])
            src2 = src_p.reshape(2, N_SUBCORES, T, 1, CHUNK)
            dst4 = dst_p.reshape(2, N_SUBCORES, T, 1, CHUNK)
            W = D
            table = h
            W_rel2 = jnp.stack([W_rel, W_rel])
        idx = jnp.concatenate([src2, dst4], axis=3)
        zeros = jnp.zeros((n_pad, W), jnp.float32)

        agg_flat = _sc_scatter_layer(table.shape[0], n_pad, T, W)(
            table, idx, zeros)
        agg2 = agg_flat.reshape(N_CORES, n_pad, W)

        head = None
        if li == 2:
            head = (lin1_W, lin1_b.reshape(1, 64), lin2_W,
                    lin2_b.reshape(1, 40))
        h = _tc_layer(agg2, h, W_rel2, W_root, b.reshape(1, D),
                      gamma.reshape(1, D), beta.reshape(1, D), n_tgt, W,
                      head=head)

    return h
